# baseline (device time: 20096 ns/iter reference)
import jax
import jax.numpy as jnp
from jax import lax
from jax.experimental import pallas as pl
from jax.experimental.pallas import tpu as pltpu

N_DEV = 4
E_PER = 2
N_EXP = N_DEV * E_PER


def kernel(x, router_W, route_idx, expert_W):
    m, d = x.shape
    e_per, _, h = expert_W.shape

    def body(x_ref, rw_ref, idx_ref, ew_ref, out_ref, comm_ref, send_sems, recv_sems):
        my = lax.axis_index("i")
        left = lax.rem(my + N_DEV - 1, N_DEV)
        right = lax.rem(my + 1, N_DEV)

        barrier_sem = pltpu.get_barrier_semaphore()
        for nbr in (left, right):
            pl.semaphore_signal(
                barrier_sem, inc=1,
                device_id=(nbr,), device_id_type=pl.DeviceIdType.MESH,
            )
        pl.semaphore_wait(barrier_sem, 2)

        comm_ref[0] = ew_ref[...]

        for s in range(N_DEV - 1):
            rdma = pltpu.make_async_remote_copy(
                src_ref=comm_ref.at[s],
                dst_ref=comm_ref.at[s + 1],
                send_sem=send_sems.at[s],
                recv_sem=recv_sems.at[s + 1],
                device_id=(right,),
                device_id_type=pl.DeviceIdType.MESH,
            )
            rdma.start()
            rdma.wait()

        xv = x_ref[...]
        scores = jnp.dot(xv, rw_ref[...], preferred_element_type=jnp.float32)
        p = jnp.exp(scores - jnp.max(scores, axis=-1, keepdims=True))
        p = p / jnp.sum(p, axis=-1, keepdims=True)
        iota8 = lax.broadcasted_iota(jnp.int32, (m, N_EXP), 1)
        m0 = (iota8 == idx_ref[:, 0:1]).astype(jnp.float32)
        m1 = (iota8 == idx_ref[:, 1:2]).astype(jnp.float32)
        g0 = jnp.sum(p * m0, axis=-1, keepdims=True)
        g1 = jnp.sum(p * m1, axis=-1, keepdims=True)
        gates = (g0 * m0 + g1 * m1) / (g0 + g1)

        acc = jnp.zeros((m, h), jnp.float32)
        for s in range(N_DEV):
            origin = lax.rem(my + N_DEV - s, N_DEV)
            for j in range(E_PER):
                e = origin * E_PER + j
                ge = jnp.sum(
                    gates * (iota8 == e).astype(jnp.float32),
                    axis=-1, keepdims=True,
                )
                acc = acc + ge * jnp.dot(
                    xv, comm_ref[s, j], preferred_element_type=jnp.float32
                )
        out_ref[...] = acc

    return pl.pallas_call(
        body,
        out_shape=jax.ShapeDtypeStruct((m, h), jnp.float32),
        in_specs=[
            pl.BlockSpec(memory_space=pltpu.VMEM),
            pl.BlockSpec(memory_space=pltpu.VMEM),
            pl.BlockSpec(memory_space=pltpu.VMEM),
            pl.BlockSpec(memory_space=pltpu.VMEM),
        ],
        out_specs=pl.BlockSpec(memory_space=pltpu.VMEM),
        scratch_shapes=[
            pltpu.VMEM((N_DEV, e_per, d, h), jnp.float32),
            pltpu.SemaphoreType.DMA((N_DEV,)),
            pltpu.SemaphoreType.DMA((N_DEV,)),
        ],
        compiler_params=pltpu.CompilerParams(collective_id=0),
    )(x, router_W, route_idx, expert_W)


# device time: 13913 ns/iter; 1.4444x vs baseline; 1.4444x over previous
import jax
import jax.numpy as jnp
from jax import lax
from jax.experimental import pallas as pl
from jax.experimental.pallas import tpu as pltpu

N_DEV = 4
E_PER = 2
N_EXP = N_DEV * E_PER


def kernel(x, router_W, route_idx, expert_W):
    m, d = x.shape
    e_per, _, h = expert_W.shape

    def body(x_ref, rw_ref, idx_ref, ew_ref, out_ref, comm_ref, send_sems, recv_sems):
        my = lax.axis_index("i")

        barrier_sem = pltpu.get_barrier_semaphore()
        for k in range(1, N_DEV):
            pl.semaphore_signal(
                barrier_sem, inc=1,
                device_id=(lax.rem(my + k, N_DEV),),
                device_id_type=pl.DeviceIdType.MESH,
            )
        pl.semaphore_wait(barrier_sem, N_DEV - 1)

        sends = []
        for k in range(1, N_DEV):
            rdma = pltpu.make_async_remote_copy(
                src_ref=ew_ref,
                dst_ref=comm_ref.at[my],
                send_sem=send_sems.at[k],
                recv_sem=recv_sems.at[my],
                device_id=(lax.rem(my + k, N_DEV),),
                device_id_type=pl.DeviceIdType.MESH,
            )
            rdma.start()
            sends.append(rdma)

        xv = x_ref[...]
        scores = jnp.dot(xv, rw_ref[...], preferred_element_type=jnp.float32)
        p = jnp.exp(scores - jnp.max(scores, axis=-1, keepdims=True))
        p = p / jnp.sum(p, axis=-1, keepdims=True)
        iota8 = lax.broadcasted_iota(jnp.int32, (m, N_EXP), 1)
        m0 = (iota8 == idx_ref[:, 0:1]).astype(jnp.float32)
        m1 = (iota8 == idx_ref[:, 1:2]).astype(jnp.float32)
        g0 = jnp.sum(p * m0, axis=-1, keepdims=True)
        g1 = jnp.sum(p * m1, axis=-1, keepdims=True)
        gates = (g0 * m0 + g1 * m1) / (g0 + g1)

        def chunk_contrib(origin, w_slot):
            c = jnp.zeros((m, h), jnp.float32)
            for j in range(E_PER):
                e = origin * E_PER + j
                ge = jnp.sum(
                    gates * (iota8 == e).astype(jnp.float32),
                    axis=-1, keepdims=True,
                )
                c = c + ge * jnp.dot(
                    xv, w_slot[j], preferred_element_type=jnp.float32
                )
            return c

        acc = chunk_contrib(my, ew_ref)
        for k in (1, 3, 2):
            o = lax.rem(my + k, N_DEV)
            recv = pltpu.make_async_remote_copy(
                src_ref=ew_ref,
                dst_ref=comm_ref.at[o],
                send_sem=send_sems.at[0],
                recv_sem=recv_sems.at[o],
                device_id=(my,),
                device_id_type=pl.DeviceIdType.MESH,
            )
            recv.wait_recv()
            acc = acc + chunk_contrib(o, comm_ref.at[o])
        out_ref[...] = acc

        for rdma in sends:
            rdma.wait_send()

    return pl.pallas_call(
        body,
        out_shape=jax.ShapeDtypeStruct((m, h), jnp.float32),
        in_specs=[
            pl.BlockSpec(memory_space=pltpu.VMEM),
            pl.BlockSpec(memory_space=pltpu.VMEM),
            pl.BlockSpec(memory_space=pltpu.VMEM),
            pl.BlockSpec(memory_space=pltpu.VMEM),
        ],
        out_specs=pl.BlockSpec(memory_space=pltpu.VMEM),
        scratch_shapes=[
            pltpu.VMEM((N_DEV, e_per, d, h), jnp.float32),
            pltpu.SemaphoreType.DMA((N_DEV,)),
            pltpu.SemaphoreType.DMA((N_DEV,)),
        ],
        compiler_params=pltpu.CompilerParams(collective_id=0),
    )(x, router_W, route_idx, expert_W)


# device time: 11088 ns/iter; 1.8124x vs baseline; 1.2548x over previous
import jax
import jax.numpy as jnp
from jax import lax
from jax.experimental import pallas as pl
from jax.experimental.pallas import tpu as pltpu

N_DEV = 4
E_PER = 2
N_EXP = N_DEV * E_PER


def kernel(x, router_W, route_idx, expert_W):
    m, d = x.shape
    e_per, _, h = expert_W.shape

    def body(x_ref, rw_ref, idx_ref, ew_ref, out_ref,
             my_bf_ref, comm_ref, send_sems, recv_sems):
        my = lax.axis_index("i")

        barrier_sem = pltpu.get_barrier_semaphore()
        for k in range(1, N_DEV):
            pl.semaphore_signal(
                barrier_sem, inc=1,
                device_id=(lax.rem(my + k, N_DEV),),
                device_id_type=pl.DeviceIdType.MESH,
            )
        pl.semaphore_wait(barrier_sem, N_DEV - 1)

        my_bf_ref[...] = ew_ref[...].astype(jnp.bfloat16)
        sends = []
        for k in range(1, N_DEV):
            rdma = pltpu.make_async_remote_copy(
                src_ref=my_bf_ref,
                dst_ref=comm_ref.at[my],
                send_sem=send_sems.at[k],
                recv_sem=recv_sems.at[my],
                device_id=(lax.rem(my + k, N_DEV),),
                device_id_type=pl.DeviceIdType.MESH,
            )
            rdma.start()
            sends.append(rdma)

        xv = x_ref[...]
        xb = xv.astype(jnp.bfloat16)
        scores = jnp.dot(xv, rw_ref[...], preferred_element_type=jnp.float32)
        p = jnp.exp(scores - jnp.max(scores, axis=-1, keepdims=True))
        p = p / jnp.sum(p, axis=-1, keepdims=True)
        iota8 = lax.broadcasted_iota(jnp.int32, (m, N_EXP), 1)
        m0 = (iota8 == idx_ref[:, 0:1]).astype(jnp.float32)
        m1 = (iota8 == idx_ref[:, 1:2]).astype(jnp.float32)
        g0 = jnp.sum(p * m0, axis=-1, keepdims=True)
        g1 = jnp.sum(p * m1, axis=-1, keepdims=True)
        gates = (g0 * m0 + g1 * m1) / (g0 + g1)

        ge = {}
        for k in range(N_DEV):
            o = lax.rem(my + k, N_DEV)
            for j in range(E_PER):
                e = o * E_PER + j
                ge[(k, j)] = jnp.sum(
                    gates * (iota8 == e).astype(jnp.float32),
                    axis=-1, keepdims=True,
                )

        def chunk_contrib(k, w_slot):
            c = jnp.zeros((m, h), jnp.float32)
            for j in range(E_PER):
                c = c + ge[(k, j)] * jnp.dot(
                    xb, w_slot[j], preferred_element_type=jnp.float32
                )
            return c

        acc = chunk_contrib(0, my_bf_ref)
        for k in (1, 3, 2):
            o = lax.rem(my + k, N_DEV)
            recv = pltpu.make_async_remote_copy(
                src_ref=my_bf_ref,
                dst_ref=comm_ref.at[o],
                send_sem=send_sems.at[0],
                recv_sem=recv_sems.at[o],
                device_id=(my,),
                device_id_type=pl.DeviceIdType.MESH,
            )
            recv.wait_recv()
            acc = acc + chunk_contrib(k, comm_ref.at[o])
        out_ref[...] = acc

        for rdma in sends:
            rdma.wait_send()

    return pl.pallas_call(
        body,
        out_shape=jax.ShapeDtypeStruct((m, h), jnp.float32),
        in_specs=[
            pl.BlockSpec(memory_space=pltpu.VMEM),
            pl.BlockSpec(memory_space=pltpu.VMEM),
            pl.BlockSpec(memory_space=pltpu.VMEM),
            pl.BlockSpec(memory_space=pltpu.VMEM),
        ],
        out_specs=pl.BlockSpec(memory_space=pltpu.VMEM),
        scratch_shapes=[
            pltpu.VMEM((e_per, d, h), jnp.bfloat16),
            pltpu.VMEM((N_DEV, e_per, d, h), jnp.bfloat16),
            pltpu.SemaphoreType.DMA((N_DEV,)),
            pltpu.SemaphoreType.DMA((N_DEV,)),
        ],
        compiler_params=pltpu.CompilerParams(collective_id=0),
    )(x, router_W, route_idx, expert_W)
